# baseline (speedup 1.0000x reference)
import jax
import jax.numpy as jnp
from jax.experimental import pallas as pl
from jax.experimental.pallas import tpu as pltpu


_BLOCK_ROWS = 256


def _slice_copy_kernel(w_ref, o_ref):
    o_ref[...] = w_ref[...]


def kernel(x, weight):
    seq_len = x.shape[-2]
    dim = weight.shape[1]
    grid = (seq_len // _BLOCK_ROWS,)
    return pl.pallas_call(
        _slice_copy_kernel,
        out_shape=jax.ShapeDtypeStruct((seq_len, dim), weight.dtype),
        grid=grid,
        in_specs=[pl.BlockSpec((_BLOCK_ROWS, dim), lambda i: (i, 0))],
        out_specs=pl.BlockSpec((_BLOCK_ROWS, dim), lambda i: (i, 0)),
    )(weight)


# speedup vs baseline: 1.1430x; 1.1430x over previous
import jax
import jax.numpy as jnp
from jax.experimental import pallas as pl
from jax.experimental.pallas import tpu as pltpu


_BLOCK_ROWS = 1024


def _slice_copy_kernel(w_ref, o_ref):
    o_ref[...] = w_ref[...]


def kernel(x, weight):
    seq_len = x.shape[-2]
    dim = weight.shape[1]
    grid = (seq_len // _BLOCK_ROWS,)
    return pl.pallas_call(
        _slice_copy_kernel,
        out_shape=jax.ShapeDtypeStruct((seq_len, dim), weight.dtype),
        grid=grid,
        in_specs=[pl.BlockSpec((_BLOCK_ROWS, dim), lambda i: (i, 0))],
        out_specs=pl.BlockSpec((_BLOCK_ROWS, dim), lambda i: (i, 0)),
    )(weight)
